# baseline (speedup 1.0000x reference)
import functools

import jax
import jax.numpy as jnp
from jax.experimental import pallas as pl
from jax.experimental.pallas import tpu as pltpu

_BF = jnp.bfloat16
_F32 = jnp.float32

_SELU_ALPHA = 1.6732632423543772848170429916717
_SELU_SCALE = 1.0507009873554804934193349852946


def _pick_bm(n):
    for c in (n // 50, n // 4, n // 2, n):
        if c and n % (2 * c) == 0 and c % 8 == 0:
            return c
    return n


def _pick_bx(n):
    for c in (n // 10, n // 2, n):
        if c and n % (2 * c) == 0 and c % 8 == 0:
            return c
    return n


def _mega_body(
    x_hbm, adj_hbm, w1_ref, w2_ref, b1_ref, b2_ref, sub_ref, fc1wt_ref,
    fc1b_ref, fuswt_ref, fusb_ref, out_ref, l1_ref,
    s1_scr, s2_scr, xa, xb, aa, ab,
    sem_xa, sem_xb, sem_aa, sem_ab, *, n, bm, bx
):
    nc1 = n // bm
    ncx = n // bx

    def x_copy(c, buf, sem):
        return pltpu.make_async_copy(
            x_hbm.at[pl.ds(c * bx, bx), :], buf, sem)

    def adj_row(c):
        return jnp.where(c < nc1, c, c - nc1) * bm

    def adj_copy(c, buf, sem):
        return pltpu.make_async_copy(
            adj_hbm.at[pl.ds(adj_row(c), bm), :], buf, sem)

    x_copy(0, xa, sem_xa).start()
    x_copy(1, xb, sem_xb).start()
    adj_copy(0, aa, sem_aa).start()
    adj_copy(1, ab, sem_ab).start()

    def p0_body(p, _):
        def one(c, buf, sem):
            x_copy(c, buf, sem).wait()
            blk = jnp.dot(buf[...].astype(_BF), w1_ref[...],
                          preferred_element_type=_F32)

            @pl.when(c + 2 < ncx)
            def _():
                x_copy(c + 2, buf, sem).start()

            return blk

        v0 = one(2 * p, xa, sem_xa)
        v1 = one(2 * p + 1, xb, sem_xb)
        s1_scr[pl.ds(2 * p * bx, 2 * bx), :] = (
            jnp.concatenate([v0, v1], axis=0).astype(_BF))
        return 0

    jax.lax.fori_loop(0, ncx // 2, p0_body, 0)

    def p1_body(p, _):
        def one(c, buf, sem):
            adj_copy(c, buf, sem).wait()
            a = buf[...].astype(_BF)
            acc = jnp.dot(a, s1_scr[...], preferred_element_type=_F32)

            @pl.when(c + 2 < 2 * nc1)
            def _():
                adj_copy(c + 2, buf, sem).start()

            h = jnp.maximum(acc + b1_ref[...], 0.0).astype(_BF)
            return jnp.dot(h, w2_ref[...], preferred_element_type=_F32)

        v0 = one(2 * p, aa, sem_aa)
        v1 = one(2 * p + 1, ab, sem_ab)
        s2_scr[pl.ds(2 * p * bm, 2 * bm), :] = (
            jnp.concatenate([v0, v1], axis=0).astype(_BF))
        return 0

    jax.lax.fori_loop(0, nc1 // 2, p1_body, 0)

    def p2_body(p, gacc):
        def one(c, buf, sem):
            adj_copy(c, buf, sem).wait()
            a = buf[...].astype(_BF)
            acc = jnp.dot(a, s2_scr[...], preferred_element_type=_F32)

            @pl.when(c + 2 < 2 * nc1)
            def _():
                adj_copy(c + 2, buf, sem).start()

            h2 = jnp.maximum(acc + b2_ref[...], 0.0)
            return jnp.sum(h2, axis=0, keepdims=True)

        c0 = nc1 + 2 * p
        return gacc + one(c0, aa, sem_aa) + one(c0 + 1, ab, sem_ab)

    nclass = s2_scr.shape[1]
    gacc = jax.lax.fori_loop(
        0, nc1 // 2, p2_body, jnp.zeros((1, nclass), _F32))

    mean_h2 = gacc / jnp.float32(n)
    g = _SELU_SCALE * jnp.where(
        mean_h2 > 0, mean_h2, _SELU_ALPHA * (jnp.exp(mean_h2) - 1.0)
    )
    x_ext = (
        jnp.dot(sub_ref[...].astype(_BF), fc1wt_ref[...],
                preferred_element_type=_F32)
        + fc1b_ref[...]
    )
    out = (
        jnp.dot(g.astype(_BF), fuswt_ref[pl.ds(0, nclass), :],
                preferred_element_type=_F32)
        + jnp.dot(x_ext.astype(_BF), fuswt_ref[pl.ds(nclass, nclass), :],
                  preferred_element_type=_F32)
        + fusb_ref[...]
    )
    m = jnp.max(out, axis=1, keepdims=True)
    e = out - m
    lse = jnp.log(jnp.sum(jnp.exp(e), axis=1, keepdims=True))
    out_ref[...] = e - lse
    l1_ref[...] = jnp.mean(
        jnp.abs(fuswt_ref[...].astype(_F32))).reshape(1, 1)


@jax.jit
def kernel(x, adj, sub_fea, W1, b1, W2, b2, fc1_W, fc1_b, fus_W, fus_b):
    n, nfeat = x.shape
    nhid = W1.shape[1]
    nclass = W2.shape[1]

    w1b = W1.astype(_BF)
    w2b = W2.astype(_BF)
    fc1wt = fc1_W.T.astype(_BF)
    fuswt = fus_W.T.astype(_BF)
    b1r = b1.reshape(1, nhid)
    b2r = b2.reshape(1, nclass)
    fc1br = fc1_b.reshape(1, nclass)
    fusbr = fus_b.reshape(1, nclass)

    bm = _pick_bm(n)
    bx = _pick_bx(n)

    logp, l1 = pl.pallas_call(
        functools.partial(_mega_body, n=n, bm=bm, bx=bx),
        in_specs=[
            pl.BlockSpec(memory_space=pl.ANY),
            pl.BlockSpec(memory_space=pl.ANY),
            pl.BlockSpec((nfeat, nhid), lambda: (0, 0)),
            pl.BlockSpec((nhid, nclass), lambda: (0, 0)),
            pl.BlockSpec((1, nhid), lambda: (0, 0)),
            pl.BlockSpec((1, nclass), lambda: (0, 0)),
            pl.BlockSpec(sub_fea.shape, lambda: (0, 0)),
            pl.BlockSpec(fc1wt.shape, lambda: (0, 0)),
            pl.BlockSpec((1, nclass), lambda: (0, 0)),
            pl.BlockSpec(fuswt.shape, lambda: (0, 0)),
            pl.BlockSpec((1, nclass), lambda: (0, 0)),
        ],
        out_specs=[
            pl.BlockSpec((1, nclass), lambda: (0, 0)),
            pl.BlockSpec((1, 1), lambda: (0, 0)),
        ],
        out_shape=[
            jax.ShapeDtypeStruct((1, nclass), _F32),
            jax.ShapeDtypeStruct((1, 1), _F32),
        ],
        scratch_shapes=[
            pltpu.VMEM((n, nhid), _BF),
            pltpu.VMEM((n, nclass), _BF),
            pltpu.VMEM((bx, nfeat), _F32),
            pltpu.VMEM((bx, nfeat), _F32),
            pltpu.VMEM((bm, n), _F32),
            pltpu.VMEM((bm, n), _F32),
            pltpu.SemaphoreType.DMA,
            pltpu.SemaphoreType.DMA,
            pltpu.SemaphoreType.DMA,
            pltpu.SemaphoreType.DMA,
        ],
        grid=(),
    )(x, adj, w1b, w2b, b1r, b2r, sub_fea, fc1wt, fc1br, fuswt, fusbr)

    return logp, l1.reshape(())
